# baseline (speedup 1.0000x reference)
import functools

import jax
import jax.numpy as jnp
from jax import lax
from jax.experimental import pallas as pl
from jax.experimental.pallas import tpu as pltpu
from jax.experimental.pallas import tpu_sc as plsc

B = 16384
D = 64
L = 16

_info = plsc.get_sparse_core_info()
_NC = _info.num_cores
_NW = _info.num_cores * _info.num_subcores
BPW = B // _NW
CH = 256
NCH = BPW // CH


def _body(da_hbm, db_hbm, embp_hbm, bias_hbm, w_hbm, b0_hbm, out_hbm,
          idx_a, idx_b, i2a, i2b, rows_a, rows_b, bia, bib, w_v, b0_v,
          out_v, sem):
    wid = lax.axis_index("s") * _NC + lax.axis_index("c")
    base = wid * BPW

    pltpu.sync_copy(da_hbm.at[pl.ds(base, BPW)], idx_a)
    pltpu.sync_copy(db_hbm.at[pl.ds(base, BPW)], idx_b)
    pltpu.sync_copy(w_hbm, w_v)
    pltpu.sync_copy(b0_hbm, b0_v)
    pltpu.async_copy(bias_hbm.at[idx_a], bia, sem).wait()
    pltpu.async_copy(bias_hbm.at[idx_b], bib, sem).wait()

    lane = jnp.arange(L, dtype=jnp.int32)

    def _splat(vec, j):
        idx = jnp.full((L, 1), j, jnp.int32)
        dnums = lax.GatherDimensionNumbers(
            offset_dims=(), collapsed_slice_dims=(0,), start_index_map=(0,))
        return lax.gather(vec, idx, dnums, (1,),
                          mode=lax.GatherScatterMode.PROMISE_IN_BOUNDS)

    wchunks = [w_v[pl.ds(c * L, L)] for c in range(D // L)]
    wtail = w_v[pl.ds(D, L)]
    w_last = _splat(wtail, 0)
    b0_vec = _splat(b0_v[pl.ds(0, L)], 0)

    for c in range(NCH):
        coff = c * CH

        def idx_body(t, carry):
            off = pl.multiple_of(t * L, L)
            i2a[pl.ds(off, L)] = lax.shift_right_logical(
                idx_a[pl.ds(coff + off, L)], 1)
            i2b[pl.ds(off, L)] = lax.shift_right_logical(
                idx_b[pl.ds(coff + off, L)], 1)
            return carry

        lax.fori_loop(0, CH // L, idx_body, 0)

        cp_a = pltpu.async_copy(embp_hbm.at[i2a], rows_a, sem)
        cp_b = pltpu.async_copy(embp_hbm.at[i2b], rows_b, sem)
        cp_a.wait()
        cp_b.wait()

        def blk_body(blk, carry):
            off = pl.multiple_of(blk * L, L)
            rowidx = blk * L + lane
            ia = idx_a[pl.ds(coff + off, L)]
            ib = idx_b[pl.ds(coff + off, L)]
            ca = (ia & 1) * D
            cb = (ib & 1) * D
            ba = bia[pl.ds(coff + off, L)]
            bb = bib[pl.ds(coff + off, L)]
            acc = (ba + bb) * w_last + b0_vec
            for cc in range(D // L):
                for j in range(L):
                    k = cc * L + j
                    av = plsc.load_gather(rows_a, [rowidx, ca + k])
                    bv = plsc.load_gather(rows_b, [rowidx, cb + k])
                    acc = acc + av * bv * _splat(wchunks[cc], j)
            res = 1.0 / (1.0 + jnp.exp(-acc))
            out_v[pl.ds(coff + off, L)] = res
            return carry

        lax.fori_loop(0, CH // L, blk_body, 0)

    pltpu.sync_copy(out_v, out_hbm.at[pl.ds(base, BPW)])


@jax.jit
def _afmp(da, db, embp, bias_flat, dense_w, dense_b):
    f = functools.partial(
        pl.kernel,
        mesh=plsc.VectorSubcoreMesh(core_axis_name="c", subcore_axis_name="s"),
        compiler_params=pltpu.CompilerParams(needs_layout_passes=False),
        out_type=jax.ShapeDtypeStruct((B,), jnp.float32),
        scratch_types=[
            pltpu.VMEM((BPW,), jnp.int32),
            pltpu.VMEM((BPW,), jnp.int32),
            pltpu.VMEM((CH,), jnp.int32),
            pltpu.VMEM((CH,), jnp.int32),
            pltpu.VMEM((CH, 2 * D), jnp.float32),
            pltpu.VMEM((CH, 2 * D), jnp.float32),
            pltpu.VMEM((BPW,), jnp.float32),
            pltpu.VMEM((BPW,), jnp.float32),
            pltpu.VMEM((D + L,), jnp.float32),
            pltpu.VMEM((L,), jnp.float32),
            pltpu.VMEM((BPW,), jnp.float32),
            pltpu.SemaphoreType.DMA,
        ],
    )(_body)
    return f(da, db, embp, bias_flat, dense_w, dense_b)


def kernel(drug_a, drug_b, emb_table, bias_table, dense_w, dense_b):
    da = drug_a.astype(jnp.int32)
    db = drug_b.astype(jnp.int32)
    embp = emb_table[:1000000].reshape(500000, 2 * D)
    w_pad = jnp.pad(dense_w.reshape(-1), (0, L - 1))
    b0_pad = jnp.pad(dense_b, (0, L - 1))
    out = _afmp(da, db, embp, bias_table.reshape(-1), w_pad, b0_pad)
    return out.reshape(B, 1)


# speedup vs baseline: 2.1129x; 2.1129x over previous
import functools

import jax
import jax.numpy as jnp
from jax import lax
from jax.experimental import pallas as pl
from jax.experimental.pallas import tpu as pltpu
from jax.experimental.pallas import tpu_sc as plsc

B = 16384
D = 64
L = 16
NB = 7812
NMAIN = NB * 128
FLAT = D * NMAIN
TILE_STRIDE = NB * 1024

_info = plsc.get_sparse_core_info()
_NC = _info.num_cores
_NW = _info.num_cores * _info.num_subcores
BPW = B // _NW
CH = 256
NCH = BPW // CH
NBLK = CH // L


def _body(da_hbm, db_hbm, flat_hbm, tail_hbm, bias_hbm, w_hbm, b0_hbm,
          out_hbm, idx_a, idx_b, ga, gb, ra, rb, bia, bib, tail_v, w_v,
          b0_v, out_v, sem):
    wid = lax.axis_index("s") * _NC + lax.axis_index("c")
    base = wid * BPW

    pltpu.sync_copy(da_hbm.at[pl.ds(base, BPW)], idx_a)
    pltpu.sync_copy(db_hbm.at[pl.ds(base, BPW)], idx_b)
    pltpu.sync_copy(w_hbm, w_v)
    pltpu.sync_copy(b0_hbm, b0_v)
    pltpu.sync_copy(tail_hbm, tail_v)
    pltpu.async_copy(bias_hbm.at[idx_a], bia, sem).wait()
    pltpu.async_copy(bias_hbm.at[idx_b], bib, sem).wait()

    lane = jnp.arange(L, dtype=jnp.int32)

    def _splat(vec, j):
        idx = jnp.full((L, 1), j, jnp.int32)
        dnums = lax.GatherDimensionNumbers(
            offset_dims=(), collapsed_slice_dims=(0,), start_index_map=(0,))
        return lax.gather(vec, idx, dnums, (1,),
                          mode=lax.GatherScatterMode.PROMISE_IN_BOUNDS)

    wchunks = [w_v[pl.ds(c * L, L)] for c in range(D // L)]
    wsp = [_splat(wchunks[k // L], k % L) for k in range(D)]
    w_last = _splat(w_v[pl.ds(D, L)], 0)
    b0_vec = _splat(b0_v[pl.ds(0, L)], 0)

    kconst = [(k >> 3) * TILE_STRIDE + (k & 7) * 128 for k in range(D)]

    for c in range(NCH):
        coff = c * CH

        def idx_body(blk, carry):
            off = pl.multiple_of(blk * L, L)
            ia = jnp.minimum(idx_a[pl.ds(coff + off, L)], NMAIN - 1)
            ib = jnp.minimum(idx_b[pl.ds(coff + off, L)], NMAIN - 1)
            basea = lax.shift_right_logical(ia, 7) * 1024 + (ia & 127)
            baseb = lax.shift_right_logical(ib, 7) * 1024 + (ib & 127)
            for k in range(D):
                ga[pl.ds(k * CH + off, L)] = basea + kconst[k]
                gb[pl.ds(k * CH + off, L)] = baseb + kconst[k]
            return carry

        lax.fori_loop(0, NBLK, idx_body, 0)

        cp_a = pltpu.async_copy(flat_hbm.at[ga], ra, sem)
        cp_b = pltpu.async_copy(flat_hbm.at[gb], rb, sem)
        cp_a.wait()
        cp_b.wait()

        def fix_body(blk, carry):
            off = pl.multiple_of(blk * L, L)
            rowidx = blk * L + lane

            def fix_one(idx_ref, rows_ref):
                ii = idx_ref[pl.ds(coff + off, L)]
                mask = ii >= NMAIN
                anyt = lax.reduce_max(mask.astype(jnp.int32), axes=(0,))

                @pl.when(anyt > 0)
                def _():
                    rowt = jnp.clip(ii - NMAIN, 0, D - 1)
                    for k in range(D):
                        tv = plsc.load_gather(
                            tail_v, [rowt, jnp.full((L,), k, jnp.int32)])
                        plsc.store_scatter(
                            rows_ref, [k * CH + off + lane], tv, mask=mask)

            fix_one(idx_a, ra)
            fix_one(idx_b, rb)
            return carry

        lax.fori_loop(0, NBLK, fix_body, 0)

        def blk_body(blk, carry):
            off = pl.multiple_of(blk * L, L)
            ba = bia[pl.ds(coff + off, L)]
            bb = bib[pl.ds(coff + off, L)]
            acc = (ba + bb) * w_last + b0_vec
            for k in range(D):
                av = ra[pl.ds(k * CH + off, L)]
                bv = rb[pl.ds(k * CH + off, L)]
                acc = acc + av * bv * wsp[k]
            res = 1.0 / (1.0 + jnp.exp(-acc))
            out_v[pl.ds(coff + off, L)] = res
            return carry

        lax.fori_loop(0, NBLK, blk_body, 0)

    pltpu.sync_copy(out_v, out_hbm.at[pl.ds(base, BPW)])


@jax.jit
def _afmp(da, db, flat, tail, bias_flat, dense_w, dense_b):
    f = functools.partial(
        pl.kernel,
        mesh=plsc.VectorSubcoreMesh(core_axis_name="c", subcore_axis_name="s"),
        compiler_params=pltpu.CompilerParams(needs_layout_passes=False),
        out_type=jax.ShapeDtypeStruct((B,), jnp.float32),
        scratch_types=[
            pltpu.VMEM((BPW,), jnp.int32),
            pltpu.VMEM((BPW,), jnp.int32),
            pltpu.VMEM((D * CH,), jnp.int32),
            pltpu.VMEM((D * CH,), jnp.int32),
            pltpu.VMEM((D * CH,), jnp.float32),
            pltpu.VMEM((D * CH,), jnp.float32),
            pltpu.VMEM((BPW,), jnp.float32),
            pltpu.VMEM((BPW,), jnp.float32),
            pltpu.VMEM((D, D), jnp.float32),
            pltpu.VMEM((D + L,), jnp.float32),
            pltpu.VMEM((L,), jnp.float32),
            pltpu.VMEM((BPW,), jnp.float32),
            pltpu.SemaphoreType.DMA,
        ],
    )(_body)
    return f(da, db, flat, tail, bias_flat, dense_w, dense_b)


def kernel(drug_a, drug_b, emb_table, bias_table, dense_w, dense_b):
    da = drug_a.astype(jnp.int32)
    db = drug_b.astype(jnp.int32)
    flat = (emb_table[:NMAIN].T.reshape(8, 8, NB, 128)
            .transpose(0, 2, 1, 3).reshape(FLAT))
    tail = emb_table[NMAIN:NMAIN + D]
    w_pad = jnp.pad(dense_w.reshape(-1), (0, L - 1))
    b0_pad = jnp.pad(dense_b, (0, L - 1))
    out = _afmp(da, db, flat, tail, bias_table.reshape(-1), w_pad, b0_pad)
    return out.reshape(B, 1)
